# baseline (speedup 1.0000x reference)
import functools

import jax
import jax.numpy as jnp
from jax import lax
from jax.experimental import pallas as pl
from jax.experimental.pallas import tpu as pltpu
from jax.experimental.pallas import tpu_sc as plsc

NUM_FIELDS = 26
VOCAB = 100000
DIM = 32
BATCH = 16384

NC = 2
NS = 16
NW = NC * NS

ROWS = BATCH * NUM_FIELDS
RPW = ROWS // NW
GROUP = 128
GPW = RPW // GROUP
CHUNK_G = 13
NCHUNK = GPW // CHUNK_G
CROWS = CHUNK_G * GROUP


def _sc_body(tbl_hbm, idx_hbm, out_hbm, idx_v, buf0, buf1, gsem, wsem):
    wid = lax.axis_index("s") * NC + lax.axis_index("c")
    g_base = wid * GPW
    r_base = wid * RPW

    pltpu.sync_copy(idx_hbm.at[pl.ds(g_base, GPW)], idx_v)

    lanes = lax.iota(jnp.int32, 16)

    def _off(i, carry):
        g = i // 8
        o = (i % 8) * 16
        p0 = (g_base + g) * GROUP + o
        f = lax.rem(p0 + lanes, NUM_FIELDS)
        idx_v[g, pl.ds(o, 16)] = idx_v[g, pl.ds(o, 16)] + f * VOCAB
        return carry

    lax.fori_loop(0, GPW * 8, _off, 0)

    bufs = (buf0, buf1)
    for c in range(NCHUNK):
        buf = bufs[c % 2]
        cps = []
        for j in range(CHUNK_G):
            g = c * CHUNK_G + j
            cps.append(
                pltpu.async_copy(
                    tbl_hbm.at[idx_v.at[g]],
                    buf.at[pl.ds(j * GROUP, GROUP)],
                    gsem,
                )
            )
        for cp in cps:
            cp.wait()
        wcp = pltpu.async_copy(
            buf, out_hbm.at[pl.ds(r_base + c * CROWS, CROWS)], wsem
        )
        wcp.wait()


@jax.jit
def _sc_embed(tbl2d, idx2d):
    mesh = plsc.VectorSubcoreMesh(
        core_axis_name="c", subcore_axis_name="s", num_cores=NC, num_subcores=NS
    )
    return pl.kernel(
        _sc_body,
        out_type=jax.ShapeDtypeStruct((ROWS, DIM), jnp.float32),
        mesh=mesh,
        scratch_types=[
            pltpu.VMEM((GPW, GROUP), jnp.int32),
            pltpu.VMEM((CROWS, DIM), jnp.float32),
            pltpu.VMEM((CROWS, DIM), jnp.float32),
            pltpu.SemaphoreType.DMA,
            pltpu.SemaphoreType.DMA,
        ],
        compiler_params=pltpu.CompilerParams(use_tc_tiling_on_sc=False),
    )(tbl2d, idx2d)


def kernel(sparse_inputs, tables):
    idx2d = sparse_inputs.astype(jnp.int32).reshape(ROWS // GROUP, GROUP)
    tbl2d = tables.reshape(NUM_FIELDS * VOCAB, DIM)
    out = _sc_embed(tbl2d, idx2d)
    return out.reshape(BATCH, NUM_FIELDS, DIM)


# speedup vs baseline: 1.0020x; 1.0020x over previous
import functools

import jax
import jax.numpy as jnp
from jax import lax
from jax.experimental import pallas as pl
from jax.experimental.pallas import tpu as pltpu
from jax.experimental.pallas import tpu_sc as plsc

NUM_FIELDS = 26
VOCAB = 100000
DIM = 32
BATCH = 16384

NC = 2
NS = 16
NW = NC * NS

ROWS = BATCH * NUM_FIELDS
RPW = ROWS // NW
GROUP = 128
GPW = RPW // GROUP
CHUNK_G = 13
NCHUNK = GPW // CHUNK_G
CROWS = CHUNK_G * GROUP


def _sc_body(tbl_hbm, idx_hbm, out_hbm, idx_v, buf0, buf1, gsem0, gsem1, wsem0, wsem1):
    wid = lax.axis_index("s") * NC + lax.axis_index("c")
    g_base = wid * GPW
    r_base = wid * RPW

    pltpu.sync_copy(idx_hbm.at[pl.ds(g_base, GPW)], idx_v)

    lanes = lax.iota(jnp.int32, 16)

    def _off(i, carry):
        g = i // 8
        o = (i % 8) * 16
        p0 = (g_base + g) * GROUP + o
        f = lax.rem(p0 + lanes, NUM_FIELDS)
        idx_v[g, pl.ds(o, 16)] = idx_v[g, pl.ds(o, 16)] + f * VOCAB
        return carry

    lax.fori_loop(0, GPW * 8, _off, 0)

    bufs = (buf0, buf1)
    gsems = (gsem0, gsem1)
    wsems = (wsem0, wsem1)
    gcps = [[], []]
    wcps = [None, None]

    def fire(c):
        buf = bufs[c % 2]
        for j in range(CHUNK_G):
            g = c * CHUNK_G + j
            gcps[c % 2].append(
                pltpu.async_copy(
                    tbl_hbm.at[idx_v.at[g]],
                    buf.at[pl.ds(j * GROUP, GROUP)],
                    gsems[c % 2],
                )
            )

    fire(0)
    for c in range(NCHUNK):
        if c + 1 < NCHUNK:
            if wcps[(c + 1) % 2] is not None:
                wcps[(c + 1) % 2].wait()
            fire(c + 1)
        for cp in gcps[c % 2]:
            cp.wait()
        gcps[c % 2] = []
        wcps[c % 2] = pltpu.async_copy(
            bufs[c % 2], out_hbm.at[pl.ds(r_base + c * CROWS, CROWS)], wsems[c % 2]
        )
    wcps[(NCHUNK - 2) % 2].wait()
    wcps[(NCHUNK - 1) % 2].wait()


@jax.jit
def _sc_embed(tbl2d, idx2d):
    mesh = plsc.VectorSubcoreMesh(
        core_axis_name="c", subcore_axis_name="s", num_cores=NC, num_subcores=NS
    )
    return pl.kernel(
        _sc_body,
        out_type=jax.ShapeDtypeStruct((ROWS, DIM), jnp.float32),
        mesh=mesh,
        scratch_types=[
            pltpu.VMEM((GPW, GROUP), jnp.int32),
            pltpu.VMEM((CROWS, DIM), jnp.float32),
            pltpu.VMEM((CROWS, DIM), jnp.float32),
            pltpu.SemaphoreType.DMA,
            pltpu.SemaphoreType.DMA,
            pltpu.SemaphoreType.DMA,
            pltpu.SemaphoreType.DMA,
        ],
        compiler_params=pltpu.CompilerParams(use_tc_tiling_on_sc=False),
    )(tbl2d, idx2d)


def kernel(sparse_inputs, tables):
    idx2d = sparse_inputs.astype(jnp.int32).reshape(ROWS // GROUP, GROUP)
    tbl2d = tables.reshape(NUM_FIELDS * VOCAB, DIM)
    out = _sc_embed(tbl2d, idx2d)
    return out.reshape(BATCH, NUM_FIELDS, DIM)


# speedup vs baseline: 3.7178x; 3.7105x over previous
import jax
import jax.numpy as jnp
from jax import lax
from jax.experimental import pallas as pl
from jax.experimental.pallas import tpu as pltpu
from jax.experimental.pallas import tpu_sc as plsc

NUM_FIELDS = 26
VOCAB = 100000
DIM = 32
BATCH = 16384

NC = 2
NS = 16
NW = NC * NS

QROWS = BATCH // 4


def _sc_body(tbl_hbm, idx_hbm, out_hbm, vec_v, idx_v, out0, out1, w0, w1):
    d = lax.axis_index("s") * NC + lax.axis_index("c")
    bufs = (out0, out1)
    sems = (w0, w1)

    def _gather_chunk(q, dst):
        def _g(i, carry):
            for u in range(4):
                o = i * 64 + u * 16
                iv = idx_v[pl.ds(q * QROWS + o, 16)]
                dst[pl.ds(o, 16)] = plsc.load_gather(vec_v, [iv])
            return carry

        lax.fori_loop(0, QROWS // 64, _g, 0)

    def _task(f, carry):
        pltpu.sync_copy(tbl_hbm.at[f, d], vec_v)
        pltpu.sync_copy(idx_hbm.at[f], idx_v)
        for q in range(4):
            buf, sem = bufs[q % 2], sems[q % 2]

            if q >= 2:
                pltpu.make_async_copy(
                    buf, out_hbm.at[0, d, pl.ds(0, QROWS)], sem
                ).wait()
            else:

                @pl.when(f > 0)
                def _():
                    pltpu.make_async_copy(
                        buf, out_hbm.at[0, d, pl.ds(0, QROWS)], sem
                    ).wait()

            _gather_chunk(q, buf)
            pltpu.async_copy(buf, out_hbm.at[f, d, pl.ds(q * QROWS, QROWS)], sem)
        return carry

    lax.fori_loop(0, NUM_FIELDS, _task, 0)
    pltpu.make_async_copy(out0, out_hbm.at[0, d, pl.ds(0, QROWS)], w0).wait()
    pltpu.make_async_copy(out1, out_hbm.at[0, d, pl.ds(0, QROWS)], w1).wait()


@jax.jit
def _sc_embed(tbl_t, idx_t):
    mesh = plsc.VectorSubcoreMesh(
        core_axis_name="c", subcore_axis_name="s", num_cores=NC, num_subcores=NS
    )
    return pl.kernel(
        _sc_body,
        out_type=jax.ShapeDtypeStruct((NUM_FIELDS, DIM, BATCH), jnp.float32),
        mesh=mesh,
        scratch_types=[
            pltpu.VMEM((VOCAB,), jnp.float32),
            pltpu.VMEM((BATCH,), jnp.int32),
            pltpu.VMEM((QROWS,), jnp.float32),
            pltpu.VMEM((QROWS,), jnp.float32),
            pltpu.SemaphoreType.DMA,
            pltpu.SemaphoreType.DMA,
        ],
        compiler_params=pltpu.CompilerParams(use_tc_tiling_on_sc=True, needs_layout_passes=False),
    )(tbl_t, idx_t)


def kernel(sparse_inputs, tables):
    idx_t = jnp.transpose(sparse_inputs.astype(jnp.int32))
    tbl_t = jnp.transpose(tables, (0, 2, 1))
    out_t = _sc_embed(tbl_t, idx_t)
    return jnp.transpose(out_t, (2, 0, 1))


# speedup vs baseline: 6.1752x; 1.6610x over previous
import jax
import jax.numpy as jnp
from jax import lax
from jax.experimental import pallas as pl
from jax.experimental.pallas import tpu as pltpu
from jax.experimental.pallas import tpu_sc as plsc

NUM_FIELDS = 26
VOCAB = 100000
DIM = 32
BATCH = 16384

NC = 2
NS = 16
NW = NC * NS

QROWS = BATCH // 4


def _sc_body(tbl_hbm, idx_hbm, out_hbm, vec_v, ib0, ib1, ob0, ob1,
             is0, is1, os0, os1):
    d = lax.axis_index("s") * NC + lax.axis_index("c")
    ibufs, isems = (ib0, ib1), (is0, is1)
    obufs, osems = (ob0, ob1), (os0, os1)

    def _fire_idx(f, q_static, parity):
        pltpu.async_copy(
            idx_hbm.at[f, pl.ds(q_static * QROWS, QROWS)],
            ibufs[parity], isems[parity],
        )

    def _gather_chunk(src_idx, dst):
        @plsc.parallel_loop(0, QROWS, step=16, unroll=8)
        def _g(o):
            iv = src_idx[pl.ds(o, 16)]
            dst[pl.ds(o, 16)] = plsc.load_gather(vec_v, [iv])

    _fire_idx(0, 0, 0)

    def _task(f, carry):
        pltpu.sync_copy(tbl_hbm.at[f, d], vec_v)
        for q in range(4):
            ip, op = q % 2, q % 2
            pltpu.make_async_copy(
                ibufs[ip], out_hbm.at[0, d, pl.ds(0, QROWS)], isems[ip]
            ).wait()
            if q < 3:
                _fire_idx(f, q + 1, (q + 1) % 2)
            else:

                @pl.when(f + 1 < NUM_FIELDS)
                def _():
                    _fire_idx(f + 1, 0, 0)

            if q >= 2:
                pltpu.make_async_copy(
                    obufs[op], out_hbm.at[0, d, pl.ds(0, QROWS)], osems[op]
                ).wait()
            else:

                @pl.when(f > 0)
                def _():
                    pltpu.make_async_copy(
                        obufs[op], out_hbm.at[0, d, pl.ds(0, QROWS)], osems[op]
                    ).wait()

            _gather_chunk(ibufs[ip], obufs[op])
            pltpu.async_copy(
                obufs[op], out_hbm.at[f, d, pl.ds(q * QROWS, QROWS)], osems[op]
            )
        return carry

    lax.fori_loop(0, NUM_FIELDS, _task, 0)
    pltpu.make_async_copy(ob0, out_hbm.at[0, d, pl.ds(0, QROWS)], os0).wait()
    pltpu.make_async_copy(ob1, out_hbm.at[0, d, pl.ds(0, QROWS)], os1).wait()


@jax.jit
def _sc_embed(tbl_t, idx_t):
    mesh = plsc.VectorSubcoreMesh(
        core_axis_name="c", subcore_axis_name="s", num_cores=NC, num_subcores=NS
    )
    return pl.kernel(
        _sc_body,
        out_type=jax.ShapeDtypeStruct((NUM_FIELDS, DIM, BATCH), jnp.float32),
        mesh=mesh,
        scratch_types=[
            pltpu.VMEM((VOCAB,), jnp.float32),
            pltpu.VMEM((QROWS,), jnp.int32),
            pltpu.VMEM((QROWS,), jnp.int32),
            pltpu.VMEM((QROWS,), jnp.float32),
            pltpu.VMEM((QROWS,), jnp.float32),
            pltpu.SemaphoreType.DMA,
            pltpu.SemaphoreType.DMA,
            pltpu.SemaphoreType.DMA,
            pltpu.SemaphoreType.DMA,
        ],
        compiler_params=pltpu.CompilerParams(
            use_tc_tiling_on_sc=True, needs_layout_passes=False
        ),
    )(tbl_t, idx_t)


def kernel(sparse_inputs, tables):
    idx_t = jnp.transpose(sparse_inputs.astype(jnp.int32))
    tbl_t = jnp.transpose(tables, (0, 2, 1))
    out_t = _sc_embed(tbl_t, idx_t)
    return jnp.transpose(out_t, (2, 0, 1))


# speedup vs baseline: 6.4865x; 1.0504x over previous
import jax
import jax.numpy as jnp
from jax import lax
from jax.experimental import pallas as pl
from jax.experimental.pallas import tpu as pltpu
from jax.experimental.pallas import tpu_sc as plsc

NUM_FIELDS = 26
VOCAB = 100000
DIM = 32
BATCH = 16384

NC = 2
NS = 16
NW = NC * NS

SPLIT = 50048
BLEN = VOCAB - SPLIT
QROWS = BATCH // 4


def _sc_body(tbl_hbm, idx_hbm, out_hbm, vec_a, vec_b, ib0, ib1, ib2, out_v,
             asem, bsem, is0, is1, is2, os0, os1, os2, os3):
    d = lax.axis_index("s") * NC + lax.axis_index("c")
    ibufs, isems = (ib0, ib1, ib2), (is0, is1, is2)
    osems = (os0, os1, os2, os3)
    lanes = lax.iota(jnp.int32, 16)

    def _fire_idx(f, q_static, b):
        pltpu.async_copy(
            idx_hbm.at[f, pl.ds(q_static * QROWS, QROWS)], ibufs[b], isems[b]
        )

    def _wait_idx(b):
        pltpu.make_async_copy(
            ibufs[b], out_hbm.at[0, d, pl.ds(0, QROWS)], isems[b]
        ).wait()

    def _drain_out(q):
        pltpu.make_async_copy(
            out_v.at[pl.ds(0, QROWS)], out_hbm.at[0, d, pl.ds(0, QROWS)],
            osems[q],
        ).wait()

    def _pass1(cq, ib):
        @plsc.parallel_loop(0, QROWS, step=16, unroll=8)
        def _p(o):
            iv = ib[pl.ds(o, 16)]
            ivc = jnp.minimum(iv, SPLIT - 1)
            out_v[pl.ds(cq * QROWS + o, 16)] = plsc.load_gather(vec_a, [ivc])

    def _pass2(cq, ib):
        @plsc.parallel_loop(0, QROWS, step=16, unroll=8)
        def _p(o):
            iv = ib[pl.ds(o, 16)]
            m = iv >= SPLIT
            ivb = jnp.maximum(iv - SPLIT, 0)
            g = plsc.load_gather(vec_b, [ivb], mask=m)
            plsc.store_scatter(out_v, [lanes + (cq * QROWS + o)], g, mask=m)

    def _wb(f, q):
        pltpu.async_copy(
            out_v.at[pl.ds(q * QROWS, QROWS)],
            out_hbm.at[f, d, pl.ds(q * QROWS, QROWS)],
            osems[q],
        )

    _fire_idx(0, 0, 0)
    pltpu.async_copy(tbl_hbm.at[0, d, pl.ds(0, SPLIT)], vec_a, asem)

    def _task(f, carry):
        pltpu.make_async_copy(
            vec_a, tbl_hbm.at[0, d, pl.ds(0, SPLIT)], asem
        ).wait()
        pltpu.async_copy(tbl_hbm.at[f, d, pl.ds(SPLIT, BLEN)], vec_b, bsem)
        _fire_idx(f, 1, 1)
        _fire_idx(f, 2, 2)

        for cq, b in ((0, 0), (1, 1), (2, 2), (3, 0)):
            _wait_idx(b)

            @pl.when(f > 0)
            def _():
                _drain_out(cq)

            _pass1(cq, ibufs[b])
            if cq == 0:
                _fire_idx(f, 3, 0)

        pltpu.make_async_copy(
            vec_b, tbl_hbm.at[0, d, pl.ds(SPLIT, BLEN)], bsem
        ).wait()

        @pl.when(f + 1 < NUM_FIELDS)
        def _():
            pltpu.async_copy(
                tbl_hbm.at[f + 1, d, pl.ds(0, SPLIT)], vec_a, asem
            )

        _pass2(3, ib0)
        _wb(f, 3)

        @pl.when(f + 1 < NUM_FIELDS)
        def _():
            _fire_idx(f + 1, 0, 0)

        _pass2(1, ib1)
        _wb(f, 1)
        _fire_idx(f, 0, 1)
        _pass2(2, ib2)
        _wb(f, 2)
        _wait_idx(1)
        _pass2(0, ib1)
        _wb(f, 0)
        return carry

    lax.fori_loop(0, NUM_FIELDS, _task, 0)
    for q in range(4):
        _drain_out(q)


@jax.jit
def _sc_embed(tbl_t, idx_t):
    mesh = plsc.VectorSubcoreMesh(
        core_axis_name="c", subcore_axis_name="s", num_cores=NC, num_subcores=NS
    )
    return pl.kernel(
        _sc_body,
        out_type=jax.ShapeDtypeStruct((NUM_FIELDS, DIM, BATCH), jnp.float32),
        mesh=mesh,
        scratch_types=[
            pltpu.VMEM((SPLIT,), jnp.float32),
            pltpu.VMEM((BLEN,), jnp.float32),
            pltpu.VMEM((QROWS,), jnp.int32),
            pltpu.VMEM((QROWS,), jnp.int32),
            pltpu.VMEM((QROWS,), jnp.int32),
            pltpu.VMEM((BATCH,), jnp.float32),
            pltpu.SemaphoreType.DMA,
            pltpu.SemaphoreType.DMA,
            pltpu.SemaphoreType.DMA,
            pltpu.SemaphoreType.DMA,
            pltpu.SemaphoreType.DMA,
            pltpu.SemaphoreType.DMA,
            pltpu.SemaphoreType.DMA,
            pltpu.SemaphoreType.DMA,
            pltpu.SemaphoreType.DMA,
        ],
        compiler_params=pltpu.CompilerParams(
            use_tc_tiling_on_sc=True, needs_layout_passes=False
        ),
    )(tbl_t, idx_t)


def kernel(sparse_inputs, tables):
    idx_t = jnp.transpose(sparse_inputs.astype(jnp.int32))
    tbl_t = jnp.transpose(tables, (0, 2, 1))
    out_t = _sc_embed(tbl_t, idx_t)
    return jnp.transpose(out_t, (2, 0, 1))
